# baseline (device time: 37854 ns/iter reference)
import jax
import jax.numpy as jnp
from jax import lax
from jax.experimental import pallas as pl
from jax.experimental.pallas import tpu as pltpu


def kernel(x, W):
    m, k = x.shape
    _, n_loc = W.shape
    n_glob = 2 * n_loc

    def body(x_ref, w_ref, out_ref, comm_ref, send_sem, recv_sem):
        my_x = lax.axis_index("x")
        my_y = lax.axis_index("y")
        partner = (1 - my_x, my_y)

        logits = jnp.dot(
            x_ref[:, :].astype(jnp.bfloat16),
            w_ref[:, :].astype(jnp.bfloat16),
            preferred_element_type=jnp.float32,
        )
        comm_ref[0, :, :] = logits.astype(jnp.bfloat16)

        barrier_sem = pltpu.get_barrier_semaphore()
        pl.semaphore_signal(
            barrier_sem, inc=1,
            device_id=partner, device_id_type=pl.DeviceIdType.MESH,
        )
        pl.semaphore_wait(barrier_sem, 1)

        rdma = pltpu.make_async_remote_copy(
            src_ref=comm_ref.at[0],
            dst_ref=comm_ref.at[1],
            send_sem=send_sem,
            recv_sem=recv_sem,
            device_id=partner,
            device_id_type=pl.DeviceIdType.MESH,
        )
        rdma.start()
        rdma.wait()

        other = comm_ref[1, :, :].astype(jnp.float32)

        row_max = jnp.maximum(
            jnp.max(logits, axis=-1, keepdims=True),
            jnp.max(other, axis=-1, keepdims=True),
        )
        e_mine = jnp.exp(logits - row_max)
        e_other = jnp.exp(other - row_max)
        denom = (
            jnp.sum(e_mine, axis=-1, keepdims=True)
            + jnp.sum(e_other, axis=-1, keepdims=True)
        )
        out_ref[:, pl.ds(my_x * n_loc, n_loc)] = e_mine / denom
        out_ref[:, pl.ds((1 - my_x) * n_loc, n_loc)] = e_other / denom

    return pl.pallas_call(
        body,
        out_shape=jax.ShapeDtypeStruct((m, n_glob), jnp.float32),
        in_specs=[
            pl.BlockSpec(memory_space=pltpu.VMEM),
            pl.BlockSpec(memory_space=pltpu.VMEM),
        ],
        out_specs=pl.BlockSpec(memory_space=pltpu.VMEM),
        scratch_shapes=[
            pltpu.VMEM((2, m, n_loc), jnp.bfloat16),
            pltpu.SemaphoreType.DMA,
            pltpu.SemaphoreType.DMA,
        ],
        compiler_params=pltpu.CompilerParams(collective_id=0),
    )(x, W)


# device time: 35737 ns/iter; 1.0592x vs baseline; 1.0592x over previous
import jax
import jax.numpy as jnp
from jax import lax
from jax.experimental import pallas as pl
from jax.experimental.pallas import tpu as pltpu

N_SLICE = 4


def kernel(x, W):
    m, k = x.shape
    _, n_loc = W.shape
    n_glob = 2 * n_loc
    n_sl = n_loc // N_SLICE

    def body(x_ref, w_ref, out_ref, comm_ref, send_sems, recv_sems):
        my_x = lax.axis_index("x")
        my_y = lax.axis_index("y")
        partner = (1 - my_x, my_y)

        barrier_sem = pltpu.get_barrier_semaphore()
        pl.semaphore_signal(
            barrier_sem, inc=1,
            device_id=partner, device_id_type=pl.DeviceIdType.MESH,
        )
        pl.semaphore_wait(barrier_sem, 1)

        x_bf = x_ref[:, :].astype(jnp.bfloat16)
        rdmas = []
        maxes = []
        sums = []
        my_base = my_x * n_loc
        oth_base = (1 - my_x) * n_loc

        for s in range(N_SLICE):
            logits_s = jnp.dot(
                x_bf,
                w_ref[:, pl.ds(s * n_sl, n_sl)].astype(jnp.bfloat16),
                preferred_element_type=jnp.float32,
            )
            comm_ref[0, s, :, :] = logits_s.astype(jnp.bfloat16)
            rdma = pltpu.make_async_remote_copy(
                src_ref=comm_ref.at[0, s],
                dst_ref=comm_ref.at[1, s],
                send_sem=send_sems.at[s],
                recv_sem=recv_sems.at[s],
                device_id=partner,
                device_id_type=pl.DeviceIdType.MESH,
            )
            rdma.start()
            rdmas.append(rdma)
            m_s = jnp.max(logits_s, axis=-1, keepdims=True)
            e_s = jnp.exp(logits_s - m_s)
            maxes.append(m_s)
            sums.append(jnp.sum(e_s, axis=-1, keepdims=True))
            out_ref[:, pl.ds(my_base + s * n_sl, n_sl)] = e_s

        for s in range(N_SLICE):
            rdmas[s].wait_recv()
            oth_s = comm_ref[1, s, :, :].astype(jnp.float32)
            m_s = jnp.max(oth_s, axis=-1, keepdims=True)
            e_s = jnp.exp(oth_s - m_s)
            maxes.append(m_s)
            sums.append(jnp.sum(e_s, axis=-1, keepdims=True))
            out_ref[:, pl.ds(oth_base + s * n_sl, n_sl)] = e_s

        big_m = maxes[0]
        for m_s in maxes[1:]:
            big_m = jnp.maximum(big_m, m_s)
        corr = [jnp.exp(m_s - big_m) for m_s in maxes]
        denom = sums[0] * corr[0]
        for s_s, c_s in zip(sums[1:], corr[1:]):
            denom = denom + s_s * c_s
        for s in range(N_SLICE):
            scale = corr[s] / denom
            col = pl.ds(my_base + s * n_sl, n_sl)
            out_ref[:, col] = out_ref[:, col] * scale
        for s in range(N_SLICE):
            scale = corr[N_SLICE + s] / denom
            col = pl.ds(oth_base + s * n_sl, n_sl)
            out_ref[:, col] = out_ref[:, col] * scale

        for s in range(N_SLICE):
            rdmas[s].wait_send()

    return pl.pallas_call(
        body,
        out_shape=jax.ShapeDtypeStruct((m, n_glob), jnp.float32),
        in_specs=[
            pl.BlockSpec(memory_space=pltpu.VMEM),
            pl.BlockSpec(memory_space=pltpu.VMEM),
        ],
        out_specs=pl.BlockSpec(memory_space=pltpu.VMEM),
        scratch_shapes=[
            pltpu.VMEM((2, N_SLICE, m, n_sl), jnp.bfloat16),
            pltpu.SemaphoreType.DMA((N_SLICE,)),
            pltpu.SemaphoreType.DMA((N_SLICE,)),
        ],
        compiler_params=pltpu.CompilerParams(collective_id=0),
    )(x, W)
